# Initial kernel scaffold; baseline (speedup 1.0000x reference)
import jax
import jax.numpy as jnp
from jax.experimental import pallas as pl


def kernel(image, masks, cat_emb):
    raise NotImplementedError("write your pallas kernel here")


# speedup vs baseline: 14.3161x; 14.3161x over previous
import functools

import jax
import jax.numpy as jnp
from jax import lax
from jax.experimental import pallas as pl
from jax.experimental.pallas import tpu as pltpu
from jax.experimental.pallas import tpu_sc as plsc

H = 512
W = 512
NM = 16
NQ = 2048
CHUNK = 16
NCH = W // CHUNK
HW = H * W
NC = 2
NS = 16
NW = NC * NS
EXPAND = 0.2
OFFSET_MAX = 128.0


def _stats_body(mask_ref, scal_ref, rcf_ref, rcb_ref, ccf_ref, ccb_ref,
                words_ref, union_ref):
    i = pl.program_id(0)
    m = mask_ref[0]

    vio = lax.broadcasted_iota(jnp.int32, (H, W), 0)
    uio = lax.broadcasted_iota(jnp.int32, (H, W), 1)
    vio_c = lax.broadcasted_iota(jnp.int32, (H, 1), 0)
    uio_r = lax.broadcasted_iota(jnp.int32, (1, W), 1)

    rows = jnp.max(m, axis=1, keepdims=True)
    cols = jnp.max(m, axis=0, keepdims=True)

    big = jnp.int32(1 << 20)
    v0 = jnp.min(jnp.where(rows > 0, vio_c, big))
    v1 = jnp.max(jnp.where(rows > 0, vio_c, -1))
    u0 = jnp.min(jnp.where(cols > 0, uio_r, big))
    u1 = jnp.max(jnp.where(cols > 0, uio_r, -1))
    empty = v1 < 0
    v0 = jnp.where(empty, 0, v0)
    v1 = jnp.where(empty, H - 1, v1)
    u0 = jnp.where(empty, 0, u0)
    u1 = jnp.where(empty, W - 1, u1)

    vlen = (v1 + 1) - v0
    ulen = (u1 + 1) - u0
    dv = (jnp.float32(EXPAND) * vlen.astype(jnp.float32)).astype(jnp.int32)
    du = (jnp.float32(EXPAND) * ulen.astype(jnp.float32)).astype(jnp.int32)
    ev0 = jnp.maximum(0, v0 - dv)
    ev1 = jnp.minimum(v1 + 1 + dv, H - 1)
    eu0 = jnp.maximum(0, u0 - du)
    eu1 = jnp.minimum(u1 + 1 + du, W - 1)

    incrop = (vio >= ev0) & (vio < ev1) & (uio >= eu0) & (uio < eu1)
    fg = jnp.where(incrop, m, 0.0)

    @pl.when(i == 0)
    def _():
        union_ref[...] = m

    @pl.when(i > 0)
    def _():
        union_ref[...] = jnp.maximum(union_ref[...], m)

    fgrcnt = jnp.sum(fg, axis=1, keepdims=True)
    le_b = (lax.broadcasted_iota(jnp.int32, (H, H), 0)
            <= lax.broadcasted_iota(jnp.int32, (H, H), 1)).astype(jnp.bfloat16)
    rcnt_i = fgrcnt.astype(jnp.int32)
    rhi = lax.shift_right_logical(rcnt_i, 4).astype(jnp.bfloat16)
    rlo = (rcnt_i & 15).astype(jnp.bfloat16)
    dims = (((0,), (0,)), ((), ()))
    hi = lax.dot_general(rhi, le_b, dims, preferred_element_type=jnp.float32)
    lo = lax.dot_general(rlo, le_b, dims, preferred_element_type=jnp.float32)
    fgrcsum = hi * 16.0 + lo
    jlane = lax.broadcasted_iota(jnp.int32, (1, H), 1)
    rows_in = jnp.clip(jlane - ev0 + 1, 0, ev1 - ev0)
    croprcsum = (rows_in * (eu1 - eu0)).astype(jnp.float32)
    rcf_ref[0] = fgrcsum.astype(jnp.int32)
    rcb_ref[0] = (croprcsum - fgrcsum).astype(jnp.int32)

    selm = (lax.broadcasted_iota(jnp.int32, (W, NCH), 0) // CHUNK
            == lax.broadcasted_iota(jnp.int32, (W, NCH), 1)).astype(jnp.bfloat16)
    ut = (lax.broadcasted_iota(jnp.int32, (NCH, NCH), 0)
          <= lax.broadcasted_iota(jnp.int32, (NCH, NCH), 1)).astype(jnp.bfloat16)

    fg_b = fg.astype(jnp.bfloat16)
    cc16 = lax.dot(fg_b, selm, preferred_element_type=jnp.float32)
    fgcc = lax.dot(cc16.astype(jnp.bfloat16), ut,
                   preferred_element_type=jnp.float32)
    ccf_ref[0] = fgcc.astype(jnp.int32)
    rowin_c = ((vio_c >= ev0) & (vio_c < ev1)).astype(jnp.float32)
    cio = lax.broadcasted_iota(jnp.int32, (H, NCH), 1)
    cropcc = (jnp.maximum(0, jnp.minimum((cio + 1) * CHUNK, eu1) - eu0)
              .astype(jnp.float32) * rowin_c)
    ccb_ref[0] = (cropcc - fgcc).astype(jnp.int32)

    pr = lax.broadcasted_iota(jnp.int32, (W, NCH), 0)
    pc = lax.broadcasted_iota(jnp.int32, (W, NCH), 1)
    packm = jnp.where(pr // CHUNK == pc,
                      lax.shift_left(jnp.int32(1), pr % CHUNK), 0)
    words_ref[0] = lax.dot(fg_b, packm.astype(jnp.bfloat16),
                           preferred_element_type=jnp.float32).astype(jnp.int32)

    fg_cnt = jnp.sum(fgrcnt).astype(jnp.int32)
    bg_cnt = (ev1 - ev0) * (eu1 - eu0) - fg_cnt
    fg_i = fg.astype(jnp.int32)
    vsum = jnp.sum(vio * fg_i)
    usum = jnp.sum(uio * fg_i)
    cntf = fg_cnt.astype(jnp.float32)
    vc = vsum.astype(jnp.float32) / cntf
    uc = usum.astype(jnp.float32) / cntf
    valid = (bg_cnt > 0).astype(jnp.float32)
    span_fg = jnp.maximum(fg_cnt, 1).astype(jnp.float32)
    span_bg = jnp.maximum(bg_cnt, 1).astype(jnp.float32)

    inv = jnp.float32(1.0 / W)
    vals = [u0.astype(jnp.float32) * inv, v0.astype(jnp.float32) * inv,
            u1.astype(jnp.float32) * inv, v1.astype(jnp.float32) * inv,
            vc, uc, valid, span_fg, span_bg,
            ev0.astype(jnp.float32), ev1.astype(jnp.float32),
            eu0.astype(jnp.float32), eu1.astype(jnp.float32)]
    lanes = lax.broadcasted_iota(jnp.int32, (1, 1, 128), 2)
    acc = jnp.zeros((1, 1, 128), jnp.float32)
    for k, val in enumerate(vals):
        acc = jnp.where(lanes == k, val, acc)
    scal_ref[...] = acc


def _run_stats(m32, interpret=False):
    return pl.pallas_call(
        _stats_body,
        grid=(NM,),
        in_specs=[pl.BlockSpec((1, H, W), lambda i: (i, 0, 0))],
        out_specs=[
            pl.BlockSpec((1, 1, 128), lambda i: (i, 0, 0)),
            pl.BlockSpec((1, 1, H), lambda i: (i, 0, 0)),
            pl.BlockSpec((1, 1, H), lambda i: (i, 0, 0)),
            pl.BlockSpec((1, H, NCH), lambda i: (i, 0, 0)),
            pl.BlockSpec((1, H, NCH), lambda i: (i, 0, 0)),
            pl.BlockSpec((1, H, NCH), lambda i: (i, 0, 0)),
            pl.BlockSpec((H, W), lambda i: (0, 0)),
        ],
        out_shape=[
            jax.ShapeDtypeStruct((NM, 1, 128), jnp.float32),
            jax.ShapeDtypeStruct((NM, 1, H), jnp.int32),
            jax.ShapeDtypeStruct((NM, 1, H), jnp.int32),
            jax.ShapeDtypeStruct((NM, H, NCH), jnp.int32),
            jax.ShapeDtypeStruct((NM, H, NCH), jnp.int32),
            jax.ShapeDtypeStruct((NM, H, NCH), jnp.int32),
            jax.ShapeDtypeStruct((H, W), jnp.float32),
        ],
        interpret=interpret,
    )(m32)


def _sample_body(rc_hbm, cc_hbm, words_hbm, hb_hbm, lb_hbm, isc_hbm, fsc_hbm,
                 img_hbm, uni_hbm, out_hbm,
                 rc, cc, wv, hbv, lbv, vbuf, pbuf, ubuf,
                 pixbuf, unibuf, outv, iscv, fscv, sem):
    wid = lax.axis_index("s") * NC + lax.axis_index("c")

    pltpu.sync_copy(rc_hbm.at[wid], rc)
    pltpu.sync_copy(cc_hbm.at[wid], cc)
    pltpu.sync_copy(words_hbm.at[wid // 2], wv)
    pltpu.sync_copy(hb_hbm.at[wid], hbv)
    pltpu.sync_copy(lb_hbm.at[wid], lbv)
    pltpu.sync_copy(isc_hbm.at[wid], iscv)
    pltpu.sync_copy(fsc_hbm.at[wid], fscv)

    span = iscv[0, :].astype(jnp.uint32)
    ev0 = iscv[1, :]
    ev1 = iscv[2, :]
    eu0 = iscv[3, :]
    eu1 = iscv[4, :]
    side = iscv[5, :]
    vc = fscv[0, :]
    uc = fscv[1, :]
    vscale = fscv[2, :]

    m1 = lax.rem(jnp.full((CHUNK,), 65536, jnp.uint32), span)
    mult = lax.rem(m1 * m1, span)

    lane = lax.iota(jnp.int32, CHUNK)

    def phase1(j, carry):
        sl = pl.ds(j * CHUNK, CHUNK)
        hb = hbv[sl]
        lb = lbv[sl]
        off = lax.rem(lax.rem(hb, span) * mult + lax.rem(lb, span), span)
        t = off.astype(jnp.int32) + 1

        pos = jnp.zeros((CHUNK,), jnp.int32)
        for s in (256, 128, 64, 32, 16, 8, 4, 2, 1):
            cand = pos + s
            val = plsc.load_gather(rc, [cand - 1])
            pos = jnp.where(val < t, cand, pos)
        v = jnp.minimum(pos, H - 1)
        excl = jnp.where(v > 0, plsc.load_gather(rc, [jnp.maximum(v - 1, 0)]), 0)
        r = t - excl

        base = v * NCH
        cpos = jnp.zeros((CHUNK,), jnp.int32)
        for s in (16, 8, 4, 2, 1):
            cand = cpos + s
            val = plsc.load_gather(cc, [base + cand - 1])
            cpos = jnp.where(val < r, cand, cpos)
        c = jnp.minimum(cpos, NCH - 1)
        excl2 = jnp.where(c > 0, plsc.load_gather(cc, [base + jnp.maximum(c - 1, 0)]), 0)
        r2 = r - excl2

        word = plsc.load_gather(wv, [base + c])
        cnt = jnp.zeros((CHUNK,), jnp.int32)
        uoff = jnp.zeros((CHUNK,), jnp.int32)
        vin = (v >= ev0) & (v < ev1)
        one = jnp.full((CHUNK,), 1, jnp.int32)
        for l in range(CHUNK):
            fgbit = lax.shift_right_logical(word, jnp.full((CHUNK,), l, jnp.int32)) & one
            u_l = c * CHUNK + l
            inc = (vin & (u_l >= eu0) & (u_l < eu1)).astype(jnp.int32)
            bit = jnp.where(side > 0, inc - fgbit, fgbit)
            cnt = cnt + bit
            uoff = uoff + (cnt < r2).astype(jnp.int32)
        u = jnp.minimum(c * CHUNK + uoff, W - 1)

        vbuf[sl] = v
        ubuf[sl] = u
        pbuf[sl] = v * W + u
        return carry

    lax.fori_loop(0, NQ // CHUNK, phase1, 0)

    copies = []
    for d in range(NQ // 128):
        dsl = pl.ds(d * 128, 128)
        copies.append(pltpu.async_copy(img_hbm.at[pbuf.at[dsl]],
                                       pixbuf.at[dsl], sem))
        copies.append(pltpu.async_copy(uni_hbm.at[pbuf.at[dsl]],
                                       unibuf.at[dsl], sem))
    for cp in copies:
        cp.wait()

    inv_off = jnp.float32(1.0 / OFFSET_MAX)
    e00 = fscv[3, :]
    e01 = fscv[4, :]
    e02 = fscv[5, :]
    e20 = fscv[6, :]
    e21 = fscv[7, :]
    e22 = fscv[8, :]

    def phase3(j, carry):
        sl = pl.ds(j * CHUNK, CHUNK)
        qv = j * CHUNK + lane
        v = vbuf[sl]
        u = ubuf[sl]

        def chan(k):
            return plsc.load_gather(pixbuf, [qv, jnp.full((CHUNK,), k, jnp.int32)])

        p0 = chan(0)
        p1 = chan(1)
        p2 = chan(2)
        uni = unibuf[sl]
        offv = (v.astype(jnp.float32) - vc) * inv_off
        offu = (u.astype(jnp.float32) - uc) * inv_off
        isbg = side > 0
        unipos = uni > 0.0
        c5 = jnp.where(isbg, jnp.where(unipos, e20, e00), 0.0)
        c6 = jnp.where(isbg, jnp.where(unipos, e21, e01), 0.0)
        c7 = jnp.where(isbg, jnp.where(unipos, e22, e02), 0.0)
        obase = qv * 8
        vals = (p0, p1, p2, offv, offu, c5, c6, c7)
        for ch in range(8):
            plsc.store_scatter(outv, [obase + ch], vals[ch] * vscale)
        return carry

    lax.fori_loop(0, NQ // CHUNK, phase3, 0)

    pltpu.sync_copy(outv, out_hbm.at[wid])


_SAMPLE_SCRATCH = [
    pltpu.VMEM((H,), jnp.int32),
    pltpu.VMEM((H * NCH,), jnp.int32),
    pltpu.VMEM((H * NCH,), jnp.int32),
    pltpu.VMEM((NQ,), jnp.uint32),
    pltpu.VMEM((NQ,), jnp.uint32),
    pltpu.VMEM((NQ,), jnp.int32),
    pltpu.VMEM((NQ,), jnp.int32),
    pltpu.VMEM((NQ,), jnp.int32),
    pltpu.VMEM((NQ, 3), jnp.float32),
    pltpu.VMEM((NQ,), jnp.float32),
    pltpu.VMEM((NQ * 8,), jnp.float32),
    pltpu.VMEM((8, CHUNK), jnp.int32),
    pltpu.VMEM((16, CHUNK), jnp.float32),
    pltpu.SemaphoreType.DMA,
]


def _run_sample(rc_all, cc_all, words, hb, lb, isc, fsc, img2, uni1,
                interpret=False):
    mesh = plsc.VectorSubcoreMesh(core_axis_name="c", subcore_axis_name="s")
    fn = functools.partial(
        pl.kernel,
        out_type=jax.ShapeDtypeStruct((NW, NQ * 8), jnp.float32),
        mesh=mesh,
        scratch_types=_SAMPLE_SCRATCH,
        compiler_params=pltpu.CompilerParams(needs_layout_passes=False,
                                             use_tc_tiling_on_sc=False),
        interpret=interpret,
    )(_sample_body)
    return fn(rc_all, cc_all, words, hb, lb, isc, fsc, img2, uni1)


def _threefry_bits():
    key = jax.random.key(0)

    def draw(w):
        k = jax.random.fold_in(key, w)
        k1, k2 = jax.random.split(k)
        return (jax.random.bits(k1, (NQ,), jnp.uint32),
                jax.random.bits(k2, (NQ,), jnp.uint32))

    return jax.vmap(draw)(jnp.arange(NW))


def kernel(image, masks, cat_emb):
    m32 = masks.astype(jnp.float32)
    scal, rcf, rcb, ccf, ccb, words, union = _run_stats(m32)

    scal = scal.reshape(NM, 128)
    xyxys = scal[:, 0:4]

    def ilv(a, b):
        return jnp.stack([a, b], axis=1).reshape((NW,) + a.shape[1:])

    ones = jnp.ones((NM,), jnp.float32)
    span_w = ilv(scal[:, 7], scal[:, 8]).astype(jnp.int32)
    ev0_w = ilv(scal[:, 9], scal[:, 9]).astype(jnp.int32)
    ev1_w = ilv(scal[:, 10], scal[:, 10]).astype(jnp.int32)
    eu0_w = ilv(scal[:, 11], scal[:, 11]).astype(jnp.int32)
    eu1_w = ilv(scal[:, 12], scal[:, 12]).astype(jnp.int32)
    side_w = jnp.tile(jnp.array([0, 1], jnp.int32), NM)
    isc = jnp.stack([span_w, ev0_w, ev1_w, eu0_w, eu1_w, side_w,
                     jnp.zeros((NW,), jnp.int32),
                     jnp.zeros((NW,), jnp.int32)], axis=1)
    isc = jnp.broadcast_to(isc[:, :, None], (NW, 8, CHUNK)).astype(jnp.int32)

    vc_w = ilv(scal[:, 4], scal[:, 4])
    uc_w = ilv(scal[:, 5], scal[:, 5])
    vscale_w = ilv(ones, scal[:, 6])
    emb = [jnp.broadcast_to(cat_emb[r, ch], (NW,))
           for r in (0, 2) for ch in (0, 1, 2)]
    frows = [vc_w, uc_w, vscale_w] + emb
    frows += [jnp.zeros((NW,), jnp.float32)] * (16 - len(frows))
    fsc = jnp.stack(frows, axis=1)
    fsc = jnp.broadcast_to(fsc[:, :, None], (NW, 16, CHUNK)).astype(jnp.float32)

    rc_all = ilv(rcf.reshape(NM, H), rcb.reshape(NM, H))
    cc_all = ilv(ccf.reshape(NM, H * NCH), ccb.reshape(NM, H * NCH))
    words2 = words.reshape(NM, H * NCH)
    img2 = image.reshape(HW, 3)
    uni1 = union.reshape(HW)

    hb, lb = _threefry_bits()

    pts = _run_sample(rc_all, cc_all, words2, hb, lb, isc, fsc, img2, uni1)
    points = pts.reshape(NM, 2, NQ, 8).reshape(NM, 2 * NQ, 8)
    return (points, xyxys)
